# Initial kernel scaffold; baseline (speedup 1.0000x reference)
import jax
import jax.numpy as jnp
from jax.experimental import pallas as pl


def kernel(x, edge_index, edge_attr, u, eb_W, eb_b, nb_W, nb_b, gb_W, gb_b, dec_W1, dec_b1, dec_W2, dec_b2):
    raise NotImplementedError("write your pallas kernel here")


# speedup vs baseline: 1.8091x; 1.8091x over previous
import functools

import jax
import jax.numpy as jnp
from jax import lax
from jax.experimental import pallas as pl
from jax.experimental.pallas import tpu as pltpu
from jax.experimental.pallas import tpu_sc as plsc

N = 10000
E = 320000
D_NODE = 128
H = 64
H2 = 128
OUT = 16

NC = 2
NS = 16
NW = NC * NS
EW = E // NW
CH = 80
NCHUNK = EW // CH

_DOT = functools.partial(jnp.dot, preferred_element_type=jnp.float32,
                         precision=lax.Precision.HIGHEST)


def _proj_body(x_ref, ws_ref, wr_ref, xs_ref, xr_ref):
    xb = x_ref[...]
    xs_ref[...] = _DOT(xb, ws_ref[...])
    xr_ref[...] = _DOT(xb, wr_ref[...])


def _proj_nodes(x, ws, wr):
    blk = 2000
    grid = N // blk
    return pl.pallas_call(
        _proj_body,
        grid=(grid,),
        in_specs=[
            pl.BlockSpec((blk, D_NODE), lambda i: (i, 0)),
            pl.BlockSpec((D_NODE, H), lambda i: (0, 0)),
            pl.BlockSpec((D_NODE, H), lambda i: (0, 0)),
        ],
        out_specs=[
            pl.BlockSpec((blk, H), lambda i: (i, 0)),
            pl.BlockSpec((blk, H), lambda i: (i, 0)),
        ],
        out_shape=[
            jax.ShapeDtypeStruct((N, H), jnp.float32),
            jax.ShapeDtypeStruct((N, H), jnp.float32),
        ],
    )(x, ws, wr)


def _edge_base_body(attr_ref, wea_ref, u_ref, wu_ref, b_ref, out_ref):
    c0 = _DOT(u_ref[...], wu_ref[...]) + b_ref[...]
    out_ref[...] = _DOT(attr_ref[...], wea_ref[...]) + c0


def _edge_base(edge_attr, wea, u, wu, b):
    blk = 2560
    grid = E // blk
    return pl.pallas_call(
        _edge_base_body,
        grid=(grid,),
        in_specs=[
            pl.BlockSpec((blk, 16), lambda i: (i, 0)),
            pl.BlockSpec((16, H), lambda i: (0, 0)),
            pl.BlockSpec((1, H), lambda i: (0, 0)),
            pl.BlockSpec((H, H), lambda i: (0, 0)),
            pl.BlockSpec((1, H), lambda i: (0, 0)),
        ],
        out_specs=pl.BlockSpec((blk, H), lambda i: (i, 0)),
        out_shape=jax.ShapeDtypeStruct((E, H), jnp.float32),
    )(edge_attr, wea, u, wu, b)


def _sc1_body(ea1_h, xs_h, xr_h, snd_h, rcv_h, zeros_h,
              e1_h, aggp_h,
              sidx, ridx, eb, gs, gr, aggsh, sem1, sem2):
    cid = lax.axis_index("c")
    sid = lax.axis_index("s")
    wid = cid * NS + sid
    base = wid * EW

    @pl.when(sid == 0)
    def _():
        pltpu.sync_copy(zeros_h, aggsh)

    plsc.subcore_barrier()

    def chunk(ci, _):
        cb = base + ci * CH
        pltpu.sync_copy(snd_h.at[pl.ds(cb, CH)], sidx)
        pltpu.sync_copy(rcv_h.at[pl.ds(cb, CH)], ridx)
        cp1 = pltpu.async_copy(xs_h.at[sidx], gs, sem1)
        cp2 = pltpu.async_copy(xr_h.at[ridx], gr, sem2)
        pltpu.sync_copy(ea1_h.at[pl.ds(cb, CH)], eb)
        cp1.wait()
        cp2.wait()

        def row(r, carry):
            for k in range(H // 16):
                sl = pl.ds(k * 16, 16)
                v = eb[r, sl] + gs[r, sl] + gr[r, sl]
                eb[r, sl] = jnp.maximum(v, 0.0)
            return carry

        lax.fori_loop(0, CH, row, 0, unroll=2)
        pltpu.sync_copy(eb, e1_h.at[pl.ds(cb, CH)])
        pltpu.sync_copy(eb, aggsh.at[ridx], add=True)
        return 0

    lax.fori_loop(0, NCHUNK, chunk, 0)
    plsc.subcore_barrier()

    @pl.when(sid == 0)
    def _():
        pltpu.sync_copy(aggsh, aggp_h.at[pl.ds(cid * N, N)])


def _sc_edge_pass1(ea1, xs, xr, senders, receivers, zeros_n):
    mesh = plsc.VectorSubcoreMesh(core_axis_name="c", subcore_axis_name="s",
                                  num_cores=NC, num_subcores=NS)
    f = functools.partial(
        pl.kernel,
        out_type=[
            jax.ShapeDtypeStruct((E, H), jnp.float32),
            jax.ShapeDtypeStruct((NC * N, H), jnp.float32),
        ],
        mesh=mesh,
        compiler_params=pltpu.CompilerParams(use_tc_tiling_on_sc=False),
        scratch_types=[
            pltpu.VMEM((CH,), jnp.int32),
            pltpu.VMEM((CH,), jnp.int32),
            pltpu.VMEM((CH, H), jnp.float32),
            pltpu.VMEM((CH, H), jnp.float32),
            pltpu.VMEM((CH, H), jnp.float32),
            pltpu.VMEM_SHARED((N, H), jnp.float32),
            pltpu.SemaphoreType.DMA,
            pltpu.SemaphoreType.DMA,
        ],
    )(_sc1_body)
    return f(ea1, xs, xr, senders, receivers, zeros_n)


def _node_body(a0_ref, a1_ref, x_ref, u_ref, wa_ref, wx_ref, wun_ref, nbb_ref,
               gbe_ref, gbn_ref, gbu_ref, gbb_ref, w1sr_ref, w1g_ref,
               db1_ref,
               nsr_ref, c2_ref,
               nsum_ref, esum_ref):
    i = pl.program_id(0)
    nblocks = pl.num_programs(0)

    @pl.when(i == 0)
    def _():
        nsum_ref[...] = jnp.zeros_like(nsum_ref)
        esum_ref[...] = jnp.zeros_like(esum_ref)

    agg = a0_ref[...] + a1_ref[...]
    esum_ref[...] += jnp.sum(agg, axis=0, keepdims=True)
    cu = _DOT(u_ref[...], wun_ref[...]) + nbb_ref[...]
    n1 = jnp.maximum(_DOT(agg, wa_ref[...]) + _DOT(x_ref[...], wx_ref[...]) + cu,
                     0.0)
    nsum_ref[...] += jnp.sum(n1, axis=0, keepdims=True)
    nsr_ref[...] = _DOT(n1, w1sr_ref[...])

    @pl.when(i == nblocks - 1)
    def _():
        e_mean = esum_ref[...] * (1.0 / E)
        n_mean = nsum_ref[...] * (1.0 / N)
        g1 = jnp.maximum(
            _DOT(e_mean, gbe_ref[...]) + _DOT(n_mean, gbn_ref[...])
            + _DOT(u_ref[...], gbu_ref[...]) + gbb_ref[...], 0.0)
        c2_ref[...] = _DOT(g1, w1g_ref[...]) + db1_ref[...]


def _node_block(aggp0, aggp1, x, u, wa, wx, wun, nbb, gbe, gbn, gbu, gbb,
                w1sr, w1g, db1):
    blk = 2000
    grid = N // blk
    full = lambda shape: pl.BlockSpec(shape, lambda i: tuple(0 for _ in shape))
    return pl.pallas_call(
        _node_body,
        grid=(grid,),
        in_specs=[
            pl.BlockSpec((blk, H), lambda i: (i, 0)),
            pl.BlockSpec((blk, H), lambda i: (i, 0)),
            pl.BlockSpec((blk, D_NODE), lambda i: (i, 0)),
            full((1, H)),
            full((H, H)), full((D_NODE, H)), full((H, H)), full((1, H)),
            full((H, H)), full((H, H)), full((H, H)), full((1, H)),
            full((H, H2)), full((H, H)), full((1, H)),
        ],
        out_specs=[
            pl.BlockSpec((blk, H2), lambda i: (i, 0)),
            pl.BlockSpec((1, H), lambda i: (0, 0)),
        ],
        out_shape=[
            jax.ShapeDtypeStruct((N, H2), jnp.float32),
            jax.ShapeDtypeStruct((1, H), jnp.float32),
        ],
        scratch_shapes=[
            pltpu.VMEM((1, H), jnp.float32),
            pltpu.VMEM((1, H), jnp.float32),
        ],
    )(aggp0, aggp1, x, u, wa, wx, wun, nbb, gbe, gbn, gbu, gbb,
      w1sr, w1g, db1)


def _sc2_body(nsr_h, snd_h, rcv_h,
              d_h,
              sidx, ridx, gs, gr, wb, sem1, sem2):
    cid = lax.axis_index("c")
    sid = lax.axis_index("s")
    wid = cid * NS + sid
    base = wid * EW

    def chunk(ci, _):
        cb = base + ci * CH
        pltpu.sync_copy(snd_h.at[pl.ds(cb, CH)], sidx)
        pltpu.sync_copy(rcv_h.at[pl.ds(cb, CH)], ridx)
        cp1 = pltpu.async_copy(nsr_h.at[sidx], gs, sem1)
        cp2 = pltpu.async_copy(nsr_h.at[ridx], gr, sem2)
        cp1.wait()
        cp2.wait()

        def row(r, carry):
            for k in range(H // 16):
                sl = pl.ds(k * 16, 16)
                wb[r, sl] = gs[r, sl] + gr[r, pl.ds(H + k * 16, 16)]
            return carry

        lax.fori_loop(0, CH, row, 0, unroll=2)
        pltpu.sync_copy(wb, d_h.at[pl.ds(cb, CH)])
        return 0

    lax.fori_loop(0, NCHUNK, chunk, 0)


def _sc_edge_pass2(nsr, senders, receivers):
    mesh = plsc.VectorSubcoreMesh(core_axis_name="c", subcore_axis_name="s",
                                  num_cores=NC, num_subcores=NS)
    f = functools.partial(
        pl.kernel,
        out_type=jax.ShapeDtypeStruct((E, H), jnp.float32),
        mesh=mesh,
        scratch_types=[
            pltpu.VMEM((CH,), jnp.int32),
            pltpu.VMEM((CH,), jnp.int32),
            pltpu.VMEM((CH, H2), jnp.float32),
            pltpu.VMEM((CH, H2), jnp.float32),
            pltpu.VMEM((CH, H), jnp.float32),
            pltpu.SemaphoreType.DMA,
            pltpu.SemaphoreType.DMA,
        ],
    )(_sc2_body)
    return f(nsr, senders, receivers)


def _dec_body(e1_ref, d_ref, c2_ref, w1e_ref, w2_ref, b2_ref, out_ref):
    p = _DOT(e1_ref[...], w1e_ref[...]) + d_ref[...] + c2_ref[...]
    out_ref[...] = _DOT(jnp.maximum(p, 0.0), w2_ref[...]) + b2_ref[...]


def _decoder(e1, d, c2, w1e, w2, b2):
    blk = 2560
    grid = E // blk
    return pl.pallas_call(
        _dec_body,
        grid=(grid,),
        in_specs=[
            pl.BlockSpec((blk, H), lambda i: (i, 0)),
            pl.BlockSpec((blk, H), lambda i: (i, 0)),
            pl.BlockSpec((1, H), lambda i: (0, 0)),
            pl.BlockSpec((H, H), lambda i: (0, 0)),
            pl.BlockSpec((H, OUT), lambda i: (0, 0)),
            pl.BlockSpec((1, OUT), lambda i: (0, 0)),
        ],
        out_specs=pl.BlockSpec((blk, OUT), lambda i: (i, 0)),
        out_shape=jax.ShapeDtypeStruct((E, OUT), jnp.float32),
    )(e1, d, c2, w1e, w2, b2)


def kernel(x, edge_index, edge_attr, u, eb_W, eb_b, nb_W, nb_b, gb_W, gb_b,
           dec_W1, dec_b1, dec_W2, dec_b2):
    senders = edge_index[0]
    receivers = edge_index[1]

    wea = eb_W[0:16]
    ws = eb_W[16:16 + D_NODE]
    wr = eb_W[16 + D_NODE:16 + 2 * D_NODE]
    wu = eb_W[16 + 2 * D_NODE:]
    wa = nb_W[0:H]
    wx = nb_W[H:H + D_NODE]
    wun = nb_W[H + D_NODE:]
    gbe = gb_W[0:H]
    gbn = gb_W[H:2 * H]
    gbu = gb_W[2 * H:]
    w1e = dec_W1[0:H]
    w1sr = jnp.concatenate([dec_W1[H:2 * H], dec_W1[2 * H:3 * H]], axis=1)
    w1g = dec_W1[3 * H:]

    u2 = u.reshape(1, H)
    ebb = eb_b.reshape(1, H)
    nbb = nb_b.reshape(1, H)
    gbb = gb_b.reshape(1, H)
    db1 = dec_b1.reshape(1, H)
    b2 = dec_b2.reshape(1, OUT)
    zeros_n = jnp.zeros((N, H), jnp.float32)

    xs, xr = _proj_nodes(x, ws, wr)
    ea1 = _edge_base(edge_attr, wea, u2, wu, ebb)
    e1, aggp = _sc_edge_pass1(ea1, xs, xr, senders, receivers, zeros_n)
    nsr, c2 = _node_block(aggp[:N], aggp[N:], x, u2, wa, wx, wun, nbb,
                          gbe, gbn, gbu, gbb, w1sr, w1g, db1)
    d = _sc_edge_pass2(nsr, senders, receivers)
    return _decoder(e1, d, c2, w1e, dec_W2, b2)


# speedup vs baseline: 2.2727x; 1.2563x over previous
import functools

import jax
import jax.numpy as jnp
from jax import lax
from jax.experimental import pallas as pl
from jax.experimental.pallas import tpu as pltpu
from jax.experimental.pallas import tpu_sc as plsc

N = 10000
E = 320000
D_NODE = 128
H = 64
H2 = 128
OUT = 16

NC = 2
NS = 16
NW = NC * NS
EW = E // NW
CH = 80
NCHUNK = EW // CH

_DOT = functools.partial(jnp.dot, preferred_element_type=jnp.float32,
                         precision=lax.Precision.DEFAULT)


def _proj_body(x_ref, ws_ref, wr_ref, xs_ref, xr_ref):
    xb = x_ref[...]
    xs_ref[...] = _DOT(xb, ws_ref[...])
    xr_ref[...] = _DOT(xb, wr_ref[...])


def _proj_nodes(x, ws, wr):
    blk = 2000
    grid = N // blk
    return pl.pallas_call(
        _proj_body,
        grid=(grid,),
        in_specs=[
            pl.BlockSpec((blk, D_NODE), lambda i: (i, 0)),
            pl.BlockSpec((D_NODE, H), lambda i: (0, 0)),
            pl.BlockSpec((D_NODE, H), lambda i: (0, 0)),
        ],
        out_specs=[
            pl.BlockSpec((blk, H), lambda i: (i, 0)),
            pl.BlockSpec((blk, H), lambda i: (i, 0)),
        ],
        out_shape=[
            jax.ShapeDtypeStruct((N, H), jnp.float32),
            jax.ShapeDtypeStruct((N, H), jnp.float32),
        ],
    )(x, ws, wr)


def _edge_base_body(attr_ref, wea_ref, u_ref, wu_ref, b_ref, out_ref):
    c0 = _DOT(u_ref[...], wu_ref[...]) + b_ref[...]
    out_ref[...] = _DOT(attr_ref[...], wea_ref[...]) + c0


def _edge_base(edge_attr, wea, u, wu, b):
    blk = 6400
    grid = E // blk
    return pl.pallas_call(
        _edge_base_body,
        grid=(grid,),
        in_specs=[
            pl.BlockSpec((blk, 16), lambda i: (i, 0)),
            pl.BlockSpec((16, H), lambda i: (0, 0)),
            pl.BlockSpec((1, H), lambda i: (0, 0)),
            pl.BlockSpec((H, H), lambda i: (0, 0)),
            pl.BlockSpec((1, H), lambda i: (0, 0)),
        ],
        out_specs=pl.BlockSpec((blk, H), lambda i: (i, 0)),
        out_shape=jax.ShapeDtypeStruct((E, H), jnp.float32),
    )(edge_attr, wea, u, wu, b)


def _sc1_body(ea1_h, xs_h, xr_h, snd_h, rcv_h, zeros_h,
              e1_h, aggp_h,
              sidx, ridx, eb, gs, gr, aggsh, sem1, sem2):
    cid = lax.axis_index("c")
    sid = lax.axis_index("s")
    wid = cid * NS + sid
    base = wid * EW

    @pl.when(sid == 0)
    def _():
        pltpu.sync_copy(zeros_h, aggsh)

    plsc.subcore_barrier()

    def chunk(ci, _):
        cb = base + ci * CH
        pltpu.sync_copy(snd_h.at[pl.ds(cb, CH)], sidx)
        pltpu.sync_copy(rcv_h.at[pl.ds(cb, CH)], ridx)
        cp1 = pltpu.async_copy(xs_h.at[sidx], gs, sem1)
        cp2 = pltpu.async_copy(xr_h.at[ridx], gr, sem2)
        pltpu.sync_copy(ea1_h.at[pl.ds(cb, CH)], eb)
        cp1.wait()
        cp2.wait()

        def row(r, carry):
            for k in range(H // 16):
                sl = pl.ds(k * 16, 16)
                v = eb[r, sl] + gs[r, sl] + gr[r, sl]
                eb[r, sl] = jnp.maximum(v, 0.0)
            return carry

        lax.fori_loop(0, CH, row, 0, unroll=2)
        pltpu.sync_copy(eb, e1_h.at[pl.ds(cb, CH)])
        pltpu.sync_copy(eb, aggsh.at[ridx], add=True)
        return 0

    lax.fori_loop(0, NCHUNK, chunk, 0)
    plsc.subcore_barrier()

    @pl.when(sid == 0)
    def _():
        pltpu.sync_copy(aggsh, aggp_h.at[pl.ds(cid * N, N)])


def _sc_edge_pass1(ea1, xs, xr, senders, receivers, zeros_n):
    mesh = plsc.VectorSubcoreMesh(core_axis_name="c", subcore_axis_name="s",
                                  num_cores=NC, num_subcores=NS)
    f = functools.partial(
        pl.kernel,
        out_type=[
            jax.ShapeDtypeStruct((E, H), jnp.float32),
            jax.ShapeDtypeStruct((NC * N, H), jnp.float32),
        ],
        mesh=mesh,
        compiler_params=pltpu.CompilerParams(use_tc_tiling_on_sc=False),
        scratch_types=[
            pltpu.VMEM((CH,), jnp.int32),
            pltpu.VMEM((CH,), jnp.int32),
            pltpu.VMEM((CH, H), jnp.float32),
            pltpu.VMEM((CH, H), jnp.float32),
            pltpu.VMEM((CH, H), jnp.float32),
            pltpu.VMEM_SHARED((N, H), jnp.float32),
            pltpu.SemaphoreType.DMA,
            pltpu.SemaphoreType.DMA,
        ],
    )(_sc1_body)
    return f(ea1, xs, xr, senders, receivers, zeros_n)


def _node_body(a0_ref, a1_ref, x_ref, u_ref, wa_ref, wx_ref, wun_ref, nbb_ref,
               gbe_ref, gbn_ref, gbu_ref, gbb_ref, w1sr_ref, w1g_ref,
               db1_ref,
               nsr_ref, c2_ref,
               nsum_ref, esum_ref):
    i = pl.program_id(0)
    nblocks = pl.num_programs(0)

    @pl.when(i == 0)
    def _():
        nsum_ref[...] = jnp.zeros_like(nsum_ref)
        esum_ref[...] = jnp.zeros_like(esum_ref)

    agg = a0_ref[...] + a1_ref[...]
    esum_ref[...] += jnp.sum(agg, axis=0, keepdims=True)
    cu = _DOT(u_ref[...], wun_ref[...]) + nbb_ref[...]
    n1 = jnp.maximum(_DOT(agg, wa_ref[...]) + _DOT(x_ref[...], wx_ref[...]) + cu,
                     0.0)
    nsum_ref[...] += jnp.sum(n1, axis=0, keepdims=True)
    nsr_ref[...] = _DOT(n1, w1sr_ref[...])

    @pl.when(i == nblocks - 1)
    def _():
        e_mean = esum_ref[...] * (1.0 / E)
        n_mean = nsum_ref[...] * (1.0 / N)
        g1 = jnp.maximum(
            _DOT(e_mean, gbe_ref[...]) + _DOT(n_mean, gbn_ref[...])
            + _DOT(u_ref[...], gbu_ref[...]) + gbb_ref[...], 0.0)
        c2_ref[...] = _DOT(g1, w1g_ref[...]) + db1_ref[...]


def _node_block(aggp0, aggp1, x, u, wa, wx, wun, nbb, gbe, gbn, gbu, gbb,
                w1sr, w1g, db1):
    blk = 2000
    grid = N // blk
    full = lambda shape: pl.BlockSpec(shape, lambda i: tuple(0 for _ in shape))
    return pl.pallas_call(
        _node_body,
        grid=(grid,),
        in_specs=[
            pl.BlockSpec((blk, H), lambda i: (i, 0)),
            pl.BlockSpec((blk, H), lambda i: (i, 0)),
            pl.BlockSpec((blk, D_NODE), lambda i: (i, 0)),
            full((1, H)),
            full((H, H)), full((D_NODE, H)), full((H, H)), full((1, H)),
            full((H, H)), full((H, H)), full((H, H)), full((1, H)),
            full((H, H2)), full((H, H)), full((1, H)),
        ],
        out_specs=[
            pl.BlockSpec((blk, H2), lambda i: (i, 0)),
            pl.BlockSpec((1, H), lambda i: (0, 0)),
        ],
        out_shape=[
            jax.ShapeDtypeStruct((N, H2), jnp.float32),
            jax.ShapeDtypeStruct((1, H), jnp.float32),
        ],
        scratch_shapes=[
            pltpu.VMEM((1, H), jnp.float32),
            pltpu.VMEM((1, H), jnp.float32),
        ],
    )(aggp0, aggp1, x, u, wa, wx, wun, nbb, gbe, gbn, gbu, gbb,
      w1sr, w1g, db1)


def _sc2_body(nsr_h, snd_h, rcv_h,
              d_h,
              sidx, ridx, gs, gr, wb, sem1, sem2):
    cid = lax.axis_index("c")
    sid = lax.axis_index("s")
    wid = cid * NS + sid
    base = wid * EW

    def chunk(ci, _):
        cb = base + ci * CH
        pltpu.sync_copy(snd_h.at[pl.ds(cb, CH)], sidx)
        pltpu.sync_copy(rcv_h.at[pl.ds(cb, CH)], ridx)
        cp1 = pltpu.async_copy(nsr_h.at[sidx], gs, sem1)
        cp2 = pltpu.async_copy(nsr_h.at[ridx], gr, sem2)
        cp1.wait()
        cp2.wait()

        def row(r, carry):
            for k in range(H // 16):
                sl = pl.ds(k * 16, 16)
                wb[r, sl] = gs[r, sl] + gr[r, pl.ds(H + k * 16, 16)]
            return carry

        lax.fori_loop(0, CH, row, 0, unroll=2)
        pltpu.sync_copy(wb, d_h.at[pl.ds(cb, CH)])
        return 0

    lax.fori_loop(0, NCHUNK, chunk, 0)


def _sc_edge_pass2(nsr, senders, receivers):
    mesh = plsc.VectorSubcoreMesh(core_axis_name="c", subcore_axis_name="s",
                                  num_cores=NC, num_subcores=NS)
    f = functools.partial(
        pl.kernel,
        out_type=jax.ShapeDtypeStruct((E, H), jnp.float32),
        mesh=mesh,
        scratch_types=[
            pltpu.VMEM((CH,), jnp.int32),
            pltpu.VMEM((CH,), jnp.int32),
            pltpu.VMEM((CH, H2), jnp.float32),
            pltpu.VMEM((CH, H2), jnp.float32),
            pltpu.VMEM((CH, H), jnp.float32),
            pltpu.SemaphoreType.DMA,
            pltpu.SemaphoreType.DMA,
        ],
    )(_sc2_body)
    return f(nsr, senders, receivers)


def _dec_body(e1_ref, d_ref, c2_ref, w1e_ref, w2_ref, b2_ref, out_ref):
    p = _DOT(e1_ref[...], w1e_ref[...]) + d_ref[...] + c2_ref[...]
    out_ref[...] = _DOT(jnp.maximum(p, 0.0), w2_ref[...]) + b2_ref[...]


def _decoder(e1, d, c2, w1e, w2, b2):
    blk = 6400
    grid = E // blk
    return pl.pallas_call(
        _dec_body,
        grid=(grid,),
        in_specs=[
            pl.BlockSpec((blk, H), lambda i: (i, 0)),
            pl.BlockSpec((blk, H), lambda i: (i, 0)),
            pl.BlockSpec((1, H), lambda i: (0, 0)),
            pl.BlockSpec((H, H), lambda i: (0, 0)),
            pl.BlockSpec((H, OUT), lambda i: (0, 0)),
            pl.BlockSpec((1, OUT), lambda i: (0, 0)),
        ],
        out_specs=pl.BlockSpec((blk, OUT), lambda i: (i, 0)),
        out_shape=jax.ShapeDtypeStruct((E, OUT), jnp.float32),
    )(e1, d, c2, w1e, w2, b2)


def kernel(x, edge_index, edge_attr, u, eb_W, eb_b, nb_W, nb_b, gb_W, gb_b,
           dec_W1, dec_b1, dec_W2, dec_b2):
    senders = edge_index[0]
    receivers = edge_index[1]

    wea = eb_W[0:16]
    ws = eb_W[16:16 + D_NODE]
    wr = eb_W[16 + D_NODE:16 + 2 * D_NODE]
    wu = eb_W[16 + 2 * D_NODE:]
    wa = nb_W[0:H]
    wx = nb_W[H:H + D_NODE]
    wun = nb_W[H + D_NODE:]
    gbe = gb_W[0:H]
    gbn = gb_W[H:2 * H]
    gbu = gb_W[2 * H:]
    w1e = dec_W1[0:H]
    w1sr = jnp.concatenate([dec_W1[H:2 * H], dec_W1[2 * H:3 * H]], axis=1)
    w1g = dec_W1[3 * H:]

    u2 = u.reshape(1, H)
    ebb = eb_b.reshape(1, H)
    nbb = nb_b.reshape(1, H)
    gbb = gb_b.reshape(1, H)
    db1 = dec_b1.reshape(1, H)
    b2 = dec_b2.reshape(1, OUT)
    zeros_n = jnp.zeros((N, H), jnp.float32)

    xs, xr = _proj_nodes(x, ws, wr)
    ea1 = _edge_base(edge_attr, wea, u2, wu, ebb)
    e1, aggp = _sc_edge_pass1(ea1, xs, xr, senders, receivers, zeros_n)
    nsr, c2 = _node_block(aggp[:N], aggp[N:], x, u2, wa, wx, wun, nbb,
                          gbe, gbn, gbu, gbb, w1sr, w1g, db1)
    d = _sc_edge_pass2(nsr, senders, receivers)
    return _decoder(e1, d, c2, w1e, dec_W2, b2)


# speedup vs baseline: 3.2549x; 1.4322x over previous
import functools

import jax
import jax.numpy as jnp
from jax import lax
from jax.experimental import pallas as pl
from jax.experimental.pallas import tpu as pltpu
from jax.experimental.pallas import tpu_sc as plsc

N = 10000
E = 320000
D_NODE = 128
H = 64
H2 = 128
OUT = 16

NC = 2
NS = 16
NW = NC * NS
EW = E // NW
CH = 80
NCHUNK = EW // CH

_DOT = functools.partial(jnp.dot, preferred_element_type=jnp.float32,
                         precision=lax.Precision.DEFAULT)


def _proj_body(x_ref, ws_ref, wr_ref, xs_ref, xr_ref):
    xb = x_ref[...]
    xs_ref[...] = _DOT(xb, ws_ref[...])
    xr_ref[...] = _DOT(xb, wr_ref[...])


def _proj_nodes(x, ws, wr):
    blk = 2000
    grid = N // blk
    return pl.pallas_call(
        _proj_body,
        grid=(grid,),
        in_specs=[
            pl.BlockSpec((blk, D_NODE), lambda i: (i, 0)),
            pl.BlockSpec((D_NODE, H), lambda i: (0, 0)),
            pl.BlockSpec((D_NODE, H), lambda i: (0, 0)),
        ],
        out_specs=[
            pl.BlockSpec((blk, H), lambda i: (i, 0)),
            pl.BlockSpec((blk, H), lambda i: (i, 0)),
        ],
        out_shape=[
            jax.ShapeDtypeStruct((N, H), jnp.float32),
            jax.ShapeDtypeStruct((N, H), jnp.float32),
        ],
    )(x, ws, wr)


def _edge_base_body(attr_ref, wea_ref, u_ref, wu_ref, b_ref, out_ref):
    c0 = _DOT(u_ref[...], wu_ref[...]) + b_ref[...]
    out_ref[...] = _DOT(attr_ref[...], wea_ref[...]) + c0


def _edge_base(edge_attr, wea, u, wu, b):
    blk = 6400
    grid = E // blk
    return pl.pallas_call(
        _edge_base_body,
        grid=(grid,),
        in_specs=[
            pl.BlockSpec((blk, 16), lambda i: (i, 0)),
            pl.BlockSpec((16, H), lambda i: (0, 0)),
            pl.BlockSpec((1, H), lambda i: (0, 0)),
            pl.BlockSpec((H, H), lambda i: (0, 0)),
            pl.BlockSpec((1, H), lambda i: (0, 0)),
        ],
        out_specs=pl.BlockSpec((blk, H), lambda i: (i, 0)),
        out_shape=jax.ShapeDtypeStruct((E, H), jnp.float32),
    )(edge_attr, wea, u, wu, b)


def _sc1_body(ea1_h, xs_h, xr_h, snd_h, rcv_h, zeros_h,
              e1_h, aggp_h,
              sall, rall,
              eb0, gs0, gr0, wb0, eb1, gs1, gr1, wb1,
              aggsh, semi0, semi1, semo0, semo1):
    cid = lax.axis_index("c")
    sid = lax.axis_index("s")
    wid = cid * NS + sid
    base = wid * EW

    @pl.when(sid == 0)
    def _():
        pltpu.sync_copy(zeros_h, aggsh)

    pltpu.sync_copy(snd_h.at[wid], sall)
    pltpu.sync_copy(rcv_h.at[wid], rall)
    plsc.subcore_barrier()

    phases = ((eb0, gs0, gr0, wb0, semi0, semo0),
              (eb1, gs1, gr1, wb1, semi1, semo1))

    def issue_in(p, c):
        eb, gs, gr, wb, semi, semo = phases[p]
        cb = base + c * CH
        pltpu.async_copy(ea1_h.at[pl.ds(cb, CH)], eb, semi)
        pltpu.async_copy(xs_h.at[sall.at[c]], gs, semi)
        pltpu.async_copy(xr_h.at[rall.at[c]], gr, semi)

    def wait_in(p, c):
        eb, gs, gr, wb, semi, semo = phases[p]
        cb = base + c * CH
        pltpu.make_async_copy(ea1_h.at[pl.ds(cb, CH)], eb, semi).wait()
        pltpu.make_async_copy(xs_h.at[sall.at[c]], gs, semi).wait()
        pltpu.make_async_copy(xr_h.at[rall.at[c]], gr, semi).wait()

    def issue_out(p, c):
        eb, gs, gr, wb, semi, semo = phases[p]
        cb = base + c * CH
        pltpu.async_copy(wb, e1_h.at[pl.ds(cb, CH)], semo)
        pltpu.sync_copy(wb, aggsh.at[rall.at[c]], add=True)

    def wait_out(p, c):
        eb, gs, gr, wb, semi, semo = phases[p]
        cb = base + c * CH
        pltpu.make_async_copy(wb, e1_h.at[pl.ds(cb, CH)], semo).wait()

    def compute(p):
        eb, gs, gr, wb, semi, semo = phases[p]

        def row(r, carry):
            for k in range(H // 16):
                sl = pl.ds(k * 16, 16)
                wb[r, sl] = jnp.maximum(eb[r, sl] + gs[r, sl] + gr[r, sl], 0.0)
            return carry

        lax.fori_loop(0, CH, row, 0, unroll=2)

    issue_in(0, 0)
    issue_in(1, 1)

    def pair(i, _):
        for p in range(2):
            c = 2 * i + p
            wait_in(p, c)

            @pl.when(c >= 2)
            def _():
                wait_out(p, c - 2)

            compute(p)
            issue_out(p, c)
            issue_in(p, jnp.minimum(c + 2, NCHUNK - 1))
        return 0

    lax.fori_loop(0, (NCHUNK - 1) // 2, pair, 0)
    c_last = NCHUNK - 1
    wait_in(0, c_last)
    wait_out(0, c_last - 2)
    compute(0)
    issue_out(0, c_last)
    wait_in(1, c_last)
    wait_out(1, c_last - 1)
    wait_out(0, c_last)

    plsc.subcore_barrier()

    @pl.when(sid == 0)
    def _():
        pltpu.sync_copy(aggsh, aggp_h.at[pl.ds(cid * N, N)])


def _sc_edge_pass1(ea1, xs, xr, snd3, rcv3, zeros_n):
    mesh = plsc.VectorSubcoreMesh(core_axis_name="c", subcore_axis_name="s",
                                  num_cores=NC, num_subcores=NS)
    f = functools.partial(
        pl.kernel,
        out_type=[
            jax.ShapeDtypeStruct((E, H), jnp.float32),
            jax.ShapeDtypeStruct((NC * N, H), jnp.float32),
        ],
        mesh=mesh,
        compiler_params=pltpu.CompilerParams(use_tc_tiling_on_sc=False),
        scratch_types=[
            pltpu.VMEM((NCHUNK, CH), jnp.int32),
            pltpu.VMEM((NCHUNK, CH), jnp.int32),
            pltpu.VMEM((CH, H), jnp.float32),
            pltpu.VMEM((CH, H), jnp.float32),
            pltpu.VMEM((CH, H), jnp.float32),
            pltpu.VMEM((CH, H), jnp.float32),
            pltpu.VMEM((CH, H), jnp.float32),
            pltpu.VMEM((CH, H), jnp.float32),
            pltpu.VMEM((CH, H), jnp.float32),
            pltpu.VMEM((CH, H), jnp.float32),
            pltpu.VMEM_SHARED((N, H), jnp.float32),
            pltpu.SemaphoreType.DMA,
            pltpu.SemaphoreType.DMA,
            pltpu.SemaphoreType.DMA,
            pltpu.SemaphoreType.DMA,
        ],
    )(_sc1_body)
    return f(ea1, xs, xr, snd3, rcv3, zeros_n)


def _node_body(a0_ref, a1_ref, x_ref, u_ref, wa_ref, wx_ref, wun_ref, nbb_ref,
               gbe_ref, gbn_ref, gbu_ref, gbb_ref, w1sr_ref, w1g_ref,
               db1_ref,
               nsr_ref, c2_ref,
               nsum_ref, esum_ref):
    i = pl.program_id(0)
    nblocks = pl.num_programs(0)

    @pl.when(i == 0)
    def _():
        nsum_ref[...] = jnp.zeros_like(nsum_ref)
        esum_ref[...] = jnp.zeros_like(esum_ref)

    agg = a0_ref[...] + a1_ref[...]
    esum_ref[...] += jnp.sum(agg, axis=0, keepdims=True)
    cu = _DOT(u_ref[...], wun_ref[...]) + nbb_ref[...]
    n1 = jnp.maximum(_DOT(agg, wa_ref[...]) + _DOT(x_ref[...], wx_ref[...]) + cu,
                     0.0)
    nsum_ref[...] += jnp.sum(n1, axis=0, keepdims=True)
    nsr_ref[...] = _DOT(n1, w1sr_ref[...])

    @pl.when(i == nblocks - 1)
    def _():
        e_mean = esum_ref[...] * (1.0 / E)
        n_mean = nsum_ref[...] * (1.0 / N)
        g1 = jnp.maximum(
            _DOT(e_mean, gbe_ref[...]) + _DOT(n_mean, gbn_ref[...])
            + _DOT(u_ref[...], gbu_ref[...]) + gbb_ref[...], 0.0)
        c2_ref[...] = _DOT(g1, w1g_ref[...]) + db1_ref[...]


def _node_block(aggp0, aggp1, x, u, wa, wx, wun, nbb, gbe, gbn, gbu, gbb,
                w1sr, w1g, db1):
    blk = 2000
    grid = N // blk
    full = lambda shape: pl.BlockSpec(shape, lambda i: tuple(0 for _ in shape))
    return pl.pallas_call(
        _node_body,
        grid=(grid,),
        in_specs=[
            pl.BlockSpec((blk, H), lambda i: (i, 0)),
            pl.BlockSpec((blk, H), lambda i: (i, 0)),
            pl.BlockSpec((blk, D_NODE), lambda i: (i, 0)),
            full((1, H)),
            full((H, H)), full((D_NODE, H)), full((H, H)), full((1, H)),
            full((H, H)), full((H, H)), full((H, H)), full((1, H)),
            full((H, H2)), full((H, H)), full((1, H)),
        ],
        out_specs=[
            pl.BlockSpec((blk, H2), lambda i: (i, 0)),
            pl.BlockSpec((1, H), lambda i: (0, 0)),
        ],
        out_shape=[
            jax.ShapeDtypeStruct((N, H2), jnp.float32),
            jax.ShapeDtypeStruct((1, H), jnp.float32),
        ],
        scratch_shapes=[
            pltpu.VMEM((1, H), jnp.float32),
            pltpu.VMEM((1, H), jnp.float32),
        ],
    )(aggp0, aggp1, x, u, wa, wx, wun, nbb, gbe, gbn, gbu, gbb,
      w1sr, w1g, db1)


def _sc2_body(nsr_h, snd_h, rcv_h,
              d_h,
              sall, rall,
              gs0, gr0, wb0, gs1, gr1, wb1,
              semi0, semi1, semo0, semo1):
    cid = lax.axis_index("c")
    sid = lax.axis_index("s")
    wid = cid * NS + sid
    base = wid * EW

    pltpu.sync_copy(snd_h.at[wid], sall)
    pltpu.sync_copy(rcv_h.at[wid], rall)

    phases = ((gs0, gr0, wb0, semi0, semo0),
              (gs1, gr1, wb1, semi1, semo1))

    def issue_in(p, c):
        gs, gr, wb, semi, semo = phases[p]
        pltpu.async_copy(nsr_h.at[sall.at[c]], gs, semi)
        pltpu.async_copy(nsr_h.at[rall.at[c]], gr, semi)

    def wait_in(p, c):
        gs, gr, wb, semi, semo = phases[p]
        pltpu.make_async_copy(nsr_h.at[sall.at[c]], gs, semi).wait()
        pltpu.make_async_copy(nsr_h.at[rall.at[c]], gr, semi).wait()

    def issue_out(p, c):
        gs, gr, wb, semi, semo = phases[p]
        pltpu.async_copy(wb, d_h.at[pl.ds(base + c * CH, CH)], semo)

    def wait_out(p, c):
        gs, gr, wb, semi, semo = phases[p]
        pltpu.make_async_copy(wb, d_h.at[pl.ds(base + c * CH, CH)], semo).wait()

    def compute(p):
        gs, gr, wb, semi, semo = phases[p]

        def row(r, carry):
            for k in range(H // 16):
                sl = pl.ds(k * 16, 16)
                wb[r, sl] = gs[r, sl] + gr[r, pl.ds(H + k * 16, 16)]
            return carry

        lax.fori_loop(0, CH, row, 0, unroll=2)

    issue_in(0, 0)
    issue_in(1, 1)

    def pair(i, _):
        for p in range(2):
            c = 2 * i + p
            wait_in(p, c)

            @pl.when(c >= 2)
            def _():
                wait_out(p, c - 2)

            compute(p)
            issue_out(p, c)
            issue_in(p, jnp.minimum(c + 2, NCHUNK - 1))
        return 0

    lax.fori_loop(0, (NCHUNK - 1) // 2, pair, 0)
    c_last = NCHUNK - 1
    wait_in(0, c_last)
    wait_out(0, c_last - 2)
    compute(0)
    issue_out(0, c_last)
    wait_in(1, c_last)
    wait_out(1, c_last - 1)
    wait_out(0, c_last)


def _sc_edge_pass2(nsr, snd3, rcv3):
    mesh = plsc.VectorSubcoreMesh(core_axis_name="c", subcore_axis_name="s",
                                  num_cores=NC, num_subcores=NS)
    f = functools.partial(
        pl.kernel,
        out_type=jax.ShapeDtypeStruct((E, H), jnp.float32),
        mesh=mesh,
        scratch_types=[
            pltpu.VMEM((NCHUNK, CH), jnp.int32),
            pltpu.VMEM((NCHUNK, CH), jnp.int32),
            pltpu.VMEM((CH, H2), jnp.float32),
            pltpu.VMEM((CH, H2), jnp.float32),
            pltpu.VMEM((CH, H), jnp.float32),
            pltpu.VMEM((CH, H2), jnp.float32),
            pltpu.VMEM((CH, H2), jnp.float32),
            pltpu.VMEM((CH, H), jnp.float32),
            pltpu.SemaphoreType.DMA,
            pltpu.SemaphoreType.DMA,
            pltpu.SemaphoreType.DMA,
            pltpu.SemaphoreType.DMA,
        ],
    )(_sc2_body)
    return f(nsr, snd3, rcv3)


def _dec_body(e1_ref, d_ref, c2_ref, w1e_ref, w2_ref, b2_ref, out_ref):
    p = _DOT(e1_ref[...], w1e_ref[...]) + d_ref[...] + c2_ref[...]
    out_ref[...] = _DOT(jnp.maximum(p, 0.0), w2_ref[...]) + b2_ref[...]


def _decoder(e1, d, c2, w1e, w2, b2):
    blk = 6400
    grid = E // blk
    return pl.pallas_call(
        _dec_body,
        grid=(grid,),
        in_specs=[
            pl.BlockSpec((blk, H), lambda i: (i, 0)),
            pl.BlockSpec((blk, H), lambda i: (i, 0)),
            pl.BlockSpec((1, H), lambda i: (0, 0)),
            pl.BlockSpec((H, H), lambda i: (0, 0)),
            pl.BlockSpec((H, OUT), lambda i: (0, 0)),
            pl.BlockSpec((1, OUT), lambda i: (0, 0)),
        ],
        out_specs=pl.BlockSpec((blk, OUT), lambda i: (i, 0)),
        out_shape=jax.ShapeDtypeStruct((E, OUT), jnp.float32),
    )(e1, d, c2, w1e, w2, b2)


def kernel(x, edge_index, edge_attr, u, eb_W, eb_b, nb_W, nb_b, gb_W, gb_b,
           dec_W1, dec_b1, dec_W2, dec_b2):
    senders = edge_index[0]
    receivers = edge_index[1]

    wea = eb_W[0:16]
    ws = eb_W[16:16 + D_NODE]
    wr = eb_W[16 + D_NODE:16 + 2 * D_NODE]
    wu = eb_W[16 + 2 * D_NODE:]
    wa = nb_W[0:H]
    wx = nb_W[H:H + D_NODE]
    wun = nb_W[H + D_NODE:]
    gbe = gb_W[0:H]
    gbn = gb_W[H:2 * H]
    gbu = gb_W[2 * H:]
    w1e = dec_W1[0:H]
    w1sr = jnp.concatenate([dec_W1[H:2 * H], dec_W1[2 * H:3 * H]], axis=1)
    w1g = dec_W1[3 * H:]

    u2 = u.reshape(1, H)
    ebb = eb_b.reshape(1, H)
    nbb = nb_b.reshape(1, H)
    gbb = gb_b.reshape(1, H)
    db1 = dec_b1.reshape(1, H)
    b2 = dec_b2.reshape(1, OUT)
    zeros_n = jnp.zeros((N, H), jnp.float32)
    snd3 = senders.reshape(NW, NCHUNK, CH)
    rcv3 = receivers.reshape(NW, NCHUNK, CH)

    xs, xr = _proj_nodes(x, ws, wr)
    ea1 = _edge_base(edge_attr, wea, u2, wu, ebb)
    e1, aggp = _sc_edge_pass1(ea1, xs, xr, snd3, rcv3, zeros_n)
    nsr, c2 = _node_block(aggp[:N], aggp[N:], x, u2, wa, wx, wun, nbb,
                          gbe, gbn, gbu, gbb, w1sr, w1g, db1)
    d = _sc_edge_pass2(nsr, snd3, rcv3)
    return _decoder(e1, d, c2, w1e, dec_W2, b2)


# speedup vs baseline: 3.5782x; 1.0993x over previous
import functools

import jax
import jax.numpy as jnp
from jax import lax
from jax.experimental import pallas as pl
from jax.experimental.pallas import tpu as pltpu
from jax.experimental.pallas import tpu_sc as plsc

N = 10000
E = 320000
D_NODE = 128
H = 64
H2 = 128
OUT = 16

NC = 2
NS = 16
NW = NC * NS
EW = E // NW
CH = 80
NCHUNK = EW // CH

_DOT = functools.partial(jnp.dot, preferred_element_type=jnp.float32,
                         precision=lax.Precision.DEFAULT)


def _proj_body(x_ref, ws_ref, wr_ref, xs_ref, xr_ref):
    xb = x_ref[...]
    xs_ref[...] = _DOT(xb, ws_ref[...])
    xr_ref[...] = _DOT(xb, wr_ref[...])


def _proj_nodes(x, ws, wr):
    blk = 2000
    grid = N // blk
    return pl.pallas_call(
        _proj_body,
        grid=(grid,),
        in_specs=[
            pl.BlockSpec((blk, D_NODE), lambda i: (i, 0)),
            pl.BlockSpec((D_NODE, H), lambda i: (0, 0)),
            pl.BlockSpec((D_NODE, H), lambda i: (0, 0)),
        ],
        out_specs=[
            pl.BlockSpec((blk, H), lambda i: (i, 0)),
            pl.BlockSpec((blk, H), lambda i: (i, 0)),
        ],
        out_shape=[
            jax.ShapeDtypeStruct((N, H), jnp.float32),
            jax.ShapeDtypeStruct((N, H), jnp.float32),
        ],
    )(x, ws, wr)


def _edge_base_body(attr2_ref, wea2_ref, u_ref, wu_ref, b_ref, out_ref):
    c0 = _DOT(u_ref[...], wu_ref[...]) + b_ref[...]
    c0p = jnp.concatenate([c0, c0], axis=1)
    out_ref[...] = _DOT(attr2_ref[...], wea2_ref[...]) + c0p


def _edge_base(attr2, wea2, u, wu, b):
    blk = 3200
    grid = (E // 2) // blk
    return pl.pallas_call(
        _edge_base_body,
        grid=(grid,),
        in_specs=[
            pl.BlockSpec((blk, 32), lambda i: (i, 0)),
            pl.BlockSpec((32, H2), lambda i: (0, 0)),
            pl.BlockSpec((1, H), lambda i: (0, 0)),
            pl.BlockSpec((H, H), lambda i: (0, 0)),
            pl.BlockSpec((1, H), lambda i: (0, 0)),
        ],
        out_specs=pl.BlockSpec((blk, H2), lambda i: (i, 0)),
        out_shape=jax.ShapeDtypeStruct((E // 2, H2), jnp.float32),
    )(attr2, wea2, u, wu, b)


def _sc1_body(ea1_h, xs_h, xr_h, snd_h, rcv_h, zeros_h,
              e1_h, aggp_h,
              sall, rall,
              eb0, gs0, gr0, wb0, eb1, gs1, gr1, wb1,
              aggsh, semi0, semi1, semo0, semo1):
    cid = lax.axis_index("c")
    sid = lax.axis_index("s")
    wid = cid * NS + sid
    base = wid * EW

    @pl.when(sid == 0)
    def _():
        pltpu.sync_copy(zeros_h, aggsh)

    pltpu.sync_copy(snd_h.at[wid], sall)
    pltpu.sync_copy(rcv_h.at[wid], rall)
    plsc.subcore_barrier()

    phases = ((eb0, gs0, gr0, wb0, semi0, semo0),
              (eb1, gs1, gr1, wb1, semi1, semo1))

    def issue_in(p, c):
        eb, gs, gr, wb, semi, semo = phases[p]
        cb = base + c * CH
        pltpu.async_copy(ea1_h.at[pl.ds(cb, CH)], eb, semi)
        pltpu.async_copy(xs_h.at[sall.at[c]], gs, semi)
        pltpu.async_copy(xr_h.at[rall.at[c]], gr, semi)

    def wait_in(p, c):
        eb, gs, gr, wb, semi, semo = phases[p]
        cb = base + c * CH
        pltpu.make_async_copy(ea1_h.at[pl.ds(cb, CH)], eb, semi).wait()
        pltpu.make_async_copy(xs_h.at[sall.at[c]], gs, semi).wait()
        pltpu.make_async_copy(xr_h.at[rall.at[c]], gr, semi).wait()

    def issue_out(p, c):
        eb, gs, gr, wb, semi, semo = phases[p]
        cb = base + c * CH
        pltpu.async_copy(wb, e1_h.at[pl.ds(cb, CH)], semo)
        pltpu.sync_copy(wb, aggsh.at[rall.at[c]], add=True)

    def wait_out(p, c):
        eb, gs, gr, wb, semi, semo = phases[p]
        cb = base + c * CH
        pltpu.make_async_copy(wb, e1_h.at[pl.ds(cb, CH)], semo).wait()

    def compute(p):
        eb, gs, gr, wb, semi, semo = phases[p]

        def row(r, carry):
            for k in range(H // 16):
                sl = pl.ds(k * 16, 16)
                wb[r, sl] = jnp.maximum(eb[r, sl] + gs[r, sl] + gr[r, sl], 0.0)
            return carry

        lax.fori_loop(0, CH, row, 0, unroll=2)

    issue_in(0, 0)
    issue_in(1, 1)

    def pair(i, _):
        for p in range(2):
            c = 2 * i + p
            wait_in(p, c)

            @pl.when(c >= 2)
            def _():
                wait_out(p, c - 2)

            compute(p)
            issue_out(p, c)
            issue_in(p, jnp.minimum(c + 2, NCHUNK - 1))
        return 0

    lax.fori_loop(0, (NCHUNK - 1) // 2, pair, 0)
    c_last = NCHUNK - 1
    wait_in(0, c_last)
    wait_out(0, c_last - 2)
    compute(0)
    issue_out(0, c_last)
    wait_in(1, c_last)
    wait_out(1, c_last - 1)
    wait_out(0, c_last)

    plsc.subcore_barrier()

    @pl.when(sid == 0)
    def _():
        pltpu.sync_copy(aggsh, aggp_h.at[pl.ds(cid * N, N)])


def _sc_edge_pass1(ea1, xs, xr, snd3, rcv3, zeros_n):
    mesh = plsc.VectorSubcoreMesh(core_axis_name="c", subcore_axis_name="s",
                                  num_cores=NC, num_subcores=NS)
    f = functools.partial(
        pl.kernel,
        out_type=[
            jax.ShapeDtypeStruct((E, H), jnp.float32),
            jax.ShapeDtypeStruct((NC * N, H), jnp.float32),
        ],
        mesh=mesh,
        compiler_params=pltpu.CompilerParams(use_tc_tiling_on_sc=False),
        scratch_types=[
            pltpu.VMEM((NCHUNK, CH), jnp.int32),
            pltpu.VMEM((NCHUNK, CH), jnp.int32),
            pltpu.VMEM((CH, H), jnp.float32),
            pltpu.VMEM((CH, H), jnp.float32),
            pltpu.VMEM((CH, H), jnp.float32),
            pltpu.VMEM((CH, H), jnp.float32),
            pltpu.VMEM((CH, H), jnp.float32),
            pltpu.VMEM((CH, H), jnp.float32),
            pltpu.VMEM((CH, H), jnp.float32),
            pltpu.VMEM((CH, H), jnp.float32),
            pltpu.VMEM_SHARED((N, H), jnp.float32),
            pltpu.SemaphoreType.DMA,
            pltpu.SemaphoreType.DMA,
            pltpu.SemaphoreType.DMA,
            pltpu.SemaphoreType.DMA,
        ],
    )(_sc1_body)
    return f(ea1, xs, xr, snd3, rcv3, zeros_n)


def _node_body(a0_ref, a1_ref, x_ref, u_ref, wa_ref, wx_ref, wun_ref, nbb_ref,
               gbe_ref, gbn_ref, gbu_ref, gbb_ref, w1sr_ref, w1g_ref,
               db1_ref,
               nsr_ref, c2_ref,
               nsum_ref, esum_ref):
    i = pl.program_id(0)
    nblocks = pl.num_programs(0)

    @pl.when(i == 0)
    def _():
        nsum_ref[...] = jnp.zeros_like(nsum_ref)
        esum_ref[...] = jnp.zeros_like(esum_ref)

    agg = a0_ref[...] + a1_ref[...]
    esum_ref[...] += jnp.sum(agg, axis=0, keepdims=True)
    cu = _DOT(u_ref[...], wun_ref[...]) + nbb_ref[...]
    n1 = jnp.maximum(_DOT(agg, wa_ref[...]) + _DOT(x_ref[...], wx_ref[...]) + cu,
                     0.0)
    nsum_ref[...] += jnp.sum(n1, axis=0, keepdims=True)
    nsr_ref[...] = _DOT(n1, w1sr_ref[...])

    @pl.when(i == nblocks - 1)
    def _():
        e_mean = esum_ref[...] * (1.0 / E)
        n_mean = nsum_ref[...] * (1.0 / N)
        g1 = jnp.maximum(
            _DOT(e_mean, gbe_ref[...]) + _DOT(n_mean, gbn_ref[...])
            + _DOT(u_ref[...], gbu_ref[...]) + gbb_ref[...], 0.0)
        c2_ref[...] = _DOT(g1, w1g_ref[...]) + db1_ref[...]


def _node_block(aggp0, aggp1, x, u, wa, wx, wun, nbb, gbe, gbn, gbu, gbb,
                w1sr, w1g, db1):
    blk = 2000
    grid = N // blk
    full = lambda shape: pl.BlockSpec(shape, lambda i: tuple(0 for _ in shape))
    return pl.pallas_call(
        _node_body,
        grid=(grid,),
        in_specs=[
            pl.BlockSpec((blk, H), lambda i: (i, 0)),
            pl.BlockSpec((blk, H), lambda i: (i, 0)),
            pl.BlockSpec((blk, D_NODE), lambda i: (i, 0)),
            full((1, H)),
            full((H, H)), full((D_NODE, H)), full((H, H)), full((1, H)),
            full((H, H)), full((H, H)), full((H, H)), full((1, H)),
            full((H, H2)), full((H, H)), full((1, H)),
        ],
        out_specs=[
            pl.BlockSpec((blk, H2), lambda i: (i, 0)),
            pl.BlockSpec((1, H), lambda i: (0, 0)),
        ],
        out_shape=[
            jax.ShapeDtypeStruct((N, H2), jnp.float32),
            jax.ShapeDtypeStruct((1, H), jnp.float32),
        ],
        scratch_shapes=[
            pltpu.VMEM((1, H), jnp.float32),
            pltpu.VMEM((1, H), jnp.float32),
        ],
    )(aggp0, aggp1, x, u, wa, wx, wun, nbb, gbe, gbn, gbu, gbb,
      w1sr, w1g, db1)


def _sc2_body(nsr_h, snd_h, rcv_h,
              d_h,
              sall, rall,
              gs0, gr0, wb0, gs1, gr1, wb1,
              semi0, semi1, semo0, semo1):
    cid = lax.axis_index("c")
    sid = lax.axis_index("s")
    wid = cid * NS + sid
    base = wid * EW

    pltpu.sync_copy(snd_h.at[wid], sall)
    pltpu.sync_copy(rcv_h.at[wid], rall)

    phases = ((gs0, gr0, wb0, semi0, semo0),
              (gs1, gr1, wb1, semi1, semo1))

    def issue_in(p, c):
        gs, gr, wb, semi, semo = phases[p]
        pltpu.async_copy(nsr_h.at[sall.at[c]], gs, semi)
        pltpu.async_copy(nsr_h.at[rall.at[c]], gr, semi)

    def wait_in(p, c):
        gs, gr, wb, semi, semo = phases[p]
        pltpu.make_async_copy(nsr_h.at[sall.at[c]], gs, semi).wait()
        pltpu.make_async_copy(nsr_h.at[rall.at[c]], gr, semi).wait()

    def issue_out(p, c):
        gs, gr, wb, semi, semo = phases[p]
        pltpu.async_copy(wb, d_h.at[pl.ds(base + c * CH, CH)], semo)

    def wait_out(p, c):
        gs, gr, wb, semi, semo = phases[p]
        pltpu.make_async_copy(wb, d_h.at[pl.ds(base + c * CH, CH)], semo).wait()

    def compute(p):
        gs, gr, wb, semi, semo = phases[p]

        def row(r, carry):
            for k in range(H // 16):
                sl = pl.ds(k * 16, 16)
                wb[r, sl] = gs[r, sl] + gr[r, pl.ds(H + k * 16, 16)]
            return carry

        lax.fori_loop(0, CH, row, 0, unroll=2)

    issue_in(0, 0)
    issue_in(1, 1)

    def pair(i, _):
        for p in range(2):
            c = 2 * i + p
            wait_in(p, c)

            @pl.when(c >= 2)
            def _():
                wait_out(p, c - 2)

            compute(p)
            issue_out(p, c)
            issue_in(p, jnp.minimum(c + 2, NCHUNK - 1))
        return 0

    lax.fori_loop(0, (NCHUNK - 1) // 2, pair, 0)
    c_last = NCHUNK - 1
    wait_in(0, c_last)
    wait_out(0, c_last - 2)
    compute(0)
    issue_out(0, c_last)
    wait_in(1, c_last)
    wait_out(1, c_last - 1)
    wait_out(0, c_last)


def _sc_edge_pass2(nsr, snd3, rcv3):
    mesh = plsc.VectorSubcoreMesh(core_axis_name="c", subcore_axis_name="s",
                                  num_cores=NC, num_subcores=NS)
    f = functools.partial(
        pl.kernel,
        out_type=jax.ShapeDtypeStruct((E, H), jnp.float32),
        mesh=mesh,
        scratch_types=[
            pltpu.VMEM((NCHUNK, CH), jnp.int32),
            pltpu.VMEM((NCHUNK, CH), jnp.int32),
            pltpu.VMEM((CH, H2), jnp.float32),
            pltpu.VMEM((CH, H2), jnp.float32),
            pltpu.VMEM((CH, H), jnp.float32),
            pltpu.VMEM((CH, H2), jnp.float32),
            pltpu.VMEM((CH, H2), jnp.float32),
            pltpu.VMEM((CH, H), jnp.float32),
            pltpu.SemaphoreType.DMA,
            pltpu.SemaphoreType.DMA,
            pltpu.SemaphoreType.DMA,
            pltpu.SemaphoreType.DMA,
        ],
    )(_sc2_body)
    return f(nsr, snd3, rcv3)


def _dec_body(e1_ref, d_ref, c2_ref, w1e_ref, w2_ref, b2_ref, out_ref):
    p = _DOT(e1_ref[...], w1e_ref[...]) + d_ref[...] + c2_ref[...]
    out_ref[...] = _DOT(jnp.maximum(p, 0.0), w2_ref[...]) + b2_ref[...]


def _decoder(e1, d, c2, w1e, w2, b2):
    blk = 6400
    grid = E // blk
    return pl.pallas_call(
        _dec_body,
        grid=(grid,),
        in_specs=[
            pl.BlockSpec((blk, H), lambda i: (i, 0)),
            pl.BlockSpec((blk, H), lambda i: (i, 0)),
            pl.BlockSpec((1, H), lambda i: (0, 0)),
            pl.BlockSpec((H, H), lambda i: (0, 0)),
            pl.BlockSpec((H, OUT), lambda i: (0, 0)),
            pl.BlockSpec((1, OUT), lambda i: (0, 0)),
        ],
        out_specs=pl.BlockSpec((blk, OUT), lambda i: (i, 0)),
        out_shape=jax.ShapeDtypeStruct((E, OUT), jnp.float32),
    )(e1, d, c2, w1e, w2, b2)


def kernel(x, edge_index, edge_attr, u, eb_W, eb_b, nb_W, nb_b, gb_W, gb_b,
           dec_W1, dec_b1, dec_W2, dec_b2):
    senders = edge_index[0]
    receivers = edge_index[1]

    wea = eb_W[0:16]
    wea2 = jnp.zeros((32, H2), jnp.float32)
    wea2 = wea2.at[0:16, 0:H].set(wea).at[16:32, H:H2].set(wea)
    ws = eb_W[16:16 + D_NODE]
    wr = eb_W[16 + D_NODE:16 + 2 * D_NODE]
    wu = eb_W[16 + 2 * D_NODE:]
    wa = nb_W[0:H]
    wx = nb_W[H:H + D_NODE]
    wun = nb_W[H + D_NODE:]
    gbe = gb_W[0:H]
    gbn = gb_W[H:2 * H]
    gbu = gb_W[2 * H:]
    w1e = dec_W1[0:H]
    w1sr = jnp.concatenate([dec_W1[H:2 * H], dec_W1[2 * H:3 * H]], axis=1)
    w1g = dec_W1[3 * H:]

    u2 = u.reshape(1, H)
    ebb = eb_b.reshape(1, H)
    nbb = nb_b.reshape(1, H)
    gbb = gb_b.reshape(1, H)
    db1 = dec_b1.reshape(1, H)
    b2 = dec_b2.reshape(1, OUT)
    zeros_n = jnp.zeros((N, H), jnp.float32)
    snd3 = senders.reshape(NW, NCHUNK, CH)
    rcv3 = receivers.reshape(NW, NCHUNK, CH)
    attr2 = edge_attr.reshape(E // 2, 32)

    xs, xr = _proj_nodes(x, ws, wr)
    ea1 = _edge_base(attr2, wea2, u2, wu, ebb).reshape(E, H)
    e1, aggp = _sc_edge_pass1(ea1, xs, xr, snd3, rcv3, zeros_n)
    nsr, c2 = _node_block(aggp[:N], aggp[N:], x, u2, wa, wx, wun, nbb,
                          gbe, gbn, gbu, gbb, w1sr, w1g, db1)
    d = _sc_edge_pass2(nsr, snd3, rcv3)
    return _decoder(e1, d, c2, w1e, dec_W2, b2)


# speedup vs baseline: 3.5877x; 1.0027x over previous
import functools

import jax
import jax.numpy as jnp
from jax import lax
from jax.experimental import pallas as pl
from jax.experimental.pallas import tpu as pltpu
from jax.experimental.pallas import tpu_sc as plsc

N = 10000
E = 320000
D_NODE = 128
H = 64
H2 = 128
OUT = 16

NC = 2
NS = 16
NW = NC * NS
EW = E // NW
CH = 80
NCHUNK = EW // CH

_DOT = functools.partial(jnp.dot, preferred_element_type=jnp.float32,
                         precision=lax.Precision.DEFAULT)


def _proj_body(x_ref, ws_ref, wr_ref, xs_ref, xr_ref):
    xb = x_ref[...]
    xs_ref[...] = _DOT(xb, ws_ref[...])
    xr_ref[...] = _DOT(xb, wr_ref[...])


def _proj_nodes(x, ws, wr):
    blk = 2000
    grid = N // blk
    return pl.pallas_call(
        _proj_body,
        grid=(grid,),
        in_specs=[
            pl.BlockSpec((blk, D_NODE), lambda i: (i, 0)),
            pl.BlockSpec((D_NODE, H), lambda i: (0, 0)),
            pl.BlockSpec((D_NODE, H), lambda i: (0, 0)),
        ],
        out_specs=[
            pl.BlockSpec((blk, H), lambda i: (i, 0)),
            pl.BlockSpec((blk, H), lambda i: (i, 0)),
        ],
        out_shape=[
            jax.ShapeDtypeStruct((N, H), jnp.float32),
            jax.ShapeDtypeStruct((N, H), jnp.float32),
        ],
    )(x, ws, wr)


def _edge_base_body(attr2_ref, wea2_ref, u_ref, wu_ref, b_ref, out_ref):
    c0 = _DOT(u_ref[...], wu_ref[...]) + b_ref[...]
    c0p = jnp.concatenate([c0, c0], axis=1)
    out_ref[...] = _DOT(attr2_ref[...], wea2_ref[...]) + c0p


def _edge_base(attr2, wea2, u, wu, b):
    blk = 3200
    grid = (E // 2) // blk
    return pl.pallas_call(
        _edge_base_body,
        grid=(grid,),
        in_specs=[
            pl.BlockSpec((blk, 32), lambda i: (i, 0)),
            pl.BlockSpec((32, H2), lambda i: (0, 0)),
            pl.BlockSpec((1, H), lambda i: (0, 0)),
            pl.BlockSpec((H, H), lambda i: (0, 0)),
            pl.BlockSpec((1, H), lambda i: (0, 0)),
        ],
        out_specs=pl.BlockSpec((blk, H2), lambda i: (i, 0)),
        out_shape=jax.ShapeDtypeStruct((E // 2, H2), jnp.float32),
    )(attr2, wea2, u, wu, b)


def _sc1_body(ea1_h, xs_h, xr_h, snd_h, rcv_h, zeros_h,
              e1_h, aggp_h,
              sall, rall,
              eb0, gs0, gr0, wb0, eb1, gs1, gr1, wb1, eb2, gs2, gr2, wb2,
              aggsh, semi0, semi1, semi2, semo0, semo1, semo2):
    cid = lax.axis_index("c")
    sid = lax.axis_index("s")
    wid = cid * NS + sid
    base = wid * EW

    @pl.when(sid == 0)
    def _():
        pltpu.sync_copy(zeros_h, aggsh)

    pltpu.sync_copy(snd_h.at[wid], sall)
    pltpu.sync_copy(rcv_h.at[wid], rall)
    plsc.subcore_barrier()

    phases = ((eb0, gs0, gr0, wb0, semi0, semo0),
              (eb1, gs1, gr1, wb1, semi1, semo1),
              (eb2, gs2, gr2, wb2, semi2, semo2))

    def issue_in(p, c):
        eb, gs, gr, wb, semi, semo = phases[p]
        cb = base + c * CH
        pltpu.async_copy(ea1_h.at[pl.ds(cb, CH)], eb, semi)
        pltpu.async_copy(xs_h.at[sall.at[c]], gs, semi)
        pltpu.async_copy(xr_h.at[rall.at[c]], gr, semi)

    def wait_in(p, c):
        eb, gs, gr, wb, semi, semo = phases[p]
        cb = base + c * CH
        pltpu.make_async_copy(ea1_h.at[pl.ds(cb, CH)], eb, semi).wait()
        pltpu.make_async_copy(xs_h.at[sall.at[c]], gs, semi).wait()
        pltpu.make_async_copy(xr_h.at[rall.at[c]], gr, semi).wait()

    def issue_out(p, c):
        eb, gs, gr, wb, semi, semo = phases[p]
        cb = base + c * CH
        pltpu.async_copy(wb, e1_h.at[pl.ds(cb, CH)], semo)
        pltpu.sync_copy(wb, aggsh.at[rall.at[c]], add=True)

    def wait_out(p, c):
        eb, gs, gr, wb, semi, semo = phases[p]
        cb = base + c * CH
        pltpu.make_async_copy(wb, e1_h.at[pl.ds(cb, CH)], semo).wait()

    def compute(p):
        eb, gs, gr, wb, semi, semo = phases[p]

        def row(r, carry):
            for k in range(H // 16):
                sl = pl.ds(k * 16, 16)
                wb[r, sl] = jnp.maximum(eb[r, sl] + gs[r, sl] + gr[r, sl], 0.0)
            return carry

        lax.fori_loop(0, CH, row, 0, unroll=2)

    issue_in(0, 0)
    issue_in(1, 1)
    issue_in(2, 2)

    def trio(i, _):
        for p in range(3):
            c = 3 * i + p
            wait_in(p, c)

            @pl.when(c >= 3)
            def _():
                wait_out(p, c - 3)

            compute(p)
            issue_in(p, jnp.minimum(c + 3, NCHUNK - 1))
            issue_out(p, c)
        return 0

    lax.fori_loop(0, NCHUNK // 3, trio, 0)
    for p, c in ((0, NCHUNK - 2), (1, NCHUNK - 1)):
        wait_in(p, c)
        wait_out(p, c - 3)
        compute(p)
        issue_out(p, c)
    wait_in(2, NCHUNK - 1)
    wait_out(2, NCHUNK - 3)
    wait_out(0, NCHUNK - 2)
    wait_out(1, NCHUNK - 1)

    plsc.subcore_barrier()

    @pl.when(sid == 0)
    def _():
        pltpu.sync_copy(aggsh, aggp_h.at[pl.ds(cid * N, N)])


def _sc_edge_pass1(ea1, xs, xr, snd3, rcv3, zeros_n):
    mesh = plsc.VectorSubcoreMesh(core_axis_name="c", subcore_axis_name="s",
                                  num_cores=NC, num_subcores=NS)
    f = functools.partial(
        pl.kernel,
        out_type=[
            jax.ShapeDtypeStruct((E, H), jnp.float32),
            jax.ShapeDtypeStruct((NC * N, H), jnp.float32),
        ],
        mesh=mesh,
        compiler_params=pltpu.CompilerParams(use_tc_tiling_on_sc=False),
        scratch_types=(
            [pltpu.VMEM((NCHUNK, CH), jnp.int32)] * 2
            + [pltpu.VMEM((CH, H), jnp.float32)] * 12
            + [pltpu.VMEM_SHARED((N, H), jnp.float32)]
            + [pltpu.SemaphoreType.DMA] * 6
        ),
    )(_sc1_body)
    return f(ea1, xs, xr, snd3, rcv3, zeros_n)


def _node_body(a0_ref, a1_ref, x_ref, u_ref, wa_ref, wx_ref, wun_ref, nbb_ref,
               gbe_ref, gbn_ref, gbu_ref, gbb_ref, w1sr_ref, w1g_ref,
               db1_ref,
               nsr_ref, c2_ref,
               nsum_ref, esum_ref):
    i = pl.program_id(0)
    nblocks = pl.num_programs(0)

    @pl.when(i == 0)
    def _():
        nsum_ref[...] = jnp.zeros_like(nsum_ref)
        esum_ref[...] = jnp.zeros_like(esum_ref)

    agg = a0_ref[...] + a1_ref[...]
    esum_ref[...] += jnp.sum(agg, axis=0, keepdims=True)
    cu = _DOT(u_ref[...], wun_ref[...]) + nbb_ref[...]
    n1 = jnp.maximum(_DOT(agg, wa_ref[...]) + _DOT(x_ref[...], wx_ref[...]) + cu,
                     0.0)
    nsum_ref[...] += jnp.sum(n1, axis=0, keepdims=True)
    nsr_ref[...] = _DOT(n1, w1sr_ref[...])

    @pl.when(i == nblocks - 1)
    def _():
        e_mean = esum_ref[...] * (1.0 / E)
        n_mean = nsum_ref[...] * (1.0 / N)
        g1 = jnp.maximum(
            _DOT(e_mean, gbe_ref[...]) + _DOT(n_mean, gbn_ref[...])
            + _DOT(u_ref[...], gbu_ref[...]) + gbb_ref[...], 0.0)
        c2_ref[...] = _DOT(g1, w1g_ref[...]) + db1_ref[...]


def _node_block(aggp0, aggp1, x, u, wa, wx, wun, nbb, gbe, gbn, gbu, gbb,
                w1sr, w1g, db1):
    blk = 2000
    grid = N // blk
    full = lambda shape: pl.BlockSpec(shape, lambda i: tuple(0 for _ in shape))
    return pl.pallas_call(
        _node_body,
        grid=(grid,),
        in_specs=[
            pl.BlockSpec((blk, H), lambda i: (i, 0)),
            pl.BlockSpec((blk, H), lambda i: (i, 0)),
            pl.BlockSpec((blk, D_NODE), lambda i: (i, 0)),
            full((1, H)),
            full((H, H)), full((D_NODE, H)), full((H, H)), full((1, H)),
            full((H, H)), full((H, H)), full((H, H)), full((1, H)),
            full((H, H2)), full((H, H)), full((1, H)),
        ],
        out_specs=[
            pl.BlockSpec((blk, H2), lambda i: (i, 0)),
            pl.BlockSpec((1, H), lambda i: (0, 0)),
        ],
        out_shape=[
            jax.ShapeDtypeStruct((N, H2), jnp.float32),
            jax.ShapeDtypeStruct((1, H), jnp.float32),
        ],
        scratch_shapes=[
            pltpu.VMEM((1, H), jnp.float32),
            pltpu.VMEM((1, H), jnp.float32),
        ],
    )(aggp0, aggp1, x, u, wa, wx, wun, nbb, gbe, gbn, gbu, gbb,
      w1sr, w1g, db1)


def _sc2_body(nsr_h, snd_h, rcv_h,
              d_h,
              sall, rall,
              gs0, gr0, wb0, gs1, gr1, wb1, gs2, gr2, wb2,
              semi0, semi1, semi2, semo0, semo1, semo2):
    cid = lax.axis_index("c")
    sid = lax.axis_index("s")
    wid = cid * NS + sid
    base = wid * EW

    pltpu.sync_copy(snd_h.at[wid], sall)
    pltpu.sync_copy(rcv_h.at[wid], rall)

    phases = ((gs0, gr0, wb0, semi0, semo0),
              (gs1, gr1, wb1, semi1, semo1),
              (gs2, gr2, wb2, semi2, semo2))

    def issue_in(p, c):
        gs, gr, wb, semi, semo = phases[p]
        pltpu.async_copy(nsr_h.at[sall.at[c]], gs, semi)
        pltpu.async_copy(nsr_h.at[rall.at[c]], gr, semi)

    def wait_in(p, c):
        gs, gr, wb, semi, semo = phases[p]
        pltpu.make_async_copy(nsr_h.at[sall.at[c]], gs, semi).wait()
        pltpu.make_async_copy(nsr_h.at[rall.at[c]], gr, semi).wait()

    def issue_out(p, c):
        gs, gr, wb, semi, semo = phases[p]
        pltpu.async_copy(wb, d_h.at[pl.ds(base + c * CH, CH)], semo)

    def wait_out(p, c):
        gs, gr, wb, semi, semo = phases[p]
        pltpu.make_async_copy(wb, d_h.at[pl.ds(base + c * CH, CH)], semo).wait()

    def compute(p):
        gs, gr, wb, semi, semo = phases[p]

        def row(r, carry):
            for k in range(H // 16):
                sl = pl.ds(k * 16, 16)
                wb[r, sl] = gs[r, sl] + gr[r, pl.ds(H + k * 16, 16)]
            return carry

        lax.fori_loop(0, CH, row, 0, unroll=2)

    issue_in(0, 0)
    issue_in(1, 1)
    issue_in(2, 2)

    def trio(i, _):
        for p in range(3):
            c = 3 * i + p
            wait_in(p, c)

            @pl.when(c >= 3)
            def _():
                wait_out(p, c - 3)

            compute(p)
            issue_in(p, jnp.minimum(c + 3, NCHUNK - 1))
            issue_out(p, c)
        return 0

    lax.fori_loop(0, NCHUNK // 3, trio, 0)
    for p, c in ((0, NCHUNK - 2), (1, NCHUNK - 1)):
        wait_in(p, c)
        wait_out(p, c - 3)
        compute(p)
        issue_out(p, c)
    wait_in(2, NCHUNK - 1)
    wait_out(2, NCHUNK - 3)
    wait_out(0, NCHUNK - 2)
    wait_out(1, NCHUNK - 1)


def _sc_edge_pass2(nsr, snd3, rcv3):
    mesh = plsc.VectorSubcoreMesh(core_axis_name="c", subcore_axis_name="s",
                                  num_cores=NC, num_subcores=NS)
    f = functools.partial(
        pl.kernel,
        out_type=jax.ShapeDtypeStruct((E, H), jnp.float32),
        mesh=mesh,
        scratch_types=(
            [pltpu.VMEM((NCHUNK, CH), jnp.int32)] * 2
            + [pltpu.VMEM((CH, H2), jnp.float32),
               pltpu.VMEM((CH, H2), jnp.float32),
               pltpu.VMEM((CH, H), jnp.float32)] * 3
            + [pltpu.SemaphoreType.DMA] * 6
        ),
    )(_sc2_body)
    return f(nsr, snd3, rcv3)


def _dec_body(e1_ref, d_ref, c2_ref, w1e_ref, w2_ref, b2_ref, out_ref):
    p = _DOT(e1_ref[...], w1e_ref[...]) + d_ref[...] + c2_ref[...]
    out_ref[...] = _DOT(jnp.maximum(p, 0.0), w2_ref[...]) + b2_ref[...]


def _decoder(e1, d, c2, w1e, w2, b2):
    blk = 6400
    grid = E // blk
    return pl.pallas_call(
        _dec_body,
        grid=(grid,),
        in_specs=[
            pl.BlockSpec((blk, H), lambda i: (i, 0)),
            pl.BlockSpec((blk, H), lambda i: (i, 0)),
            pl.BlockSpec((1, H), lambda i: (0, 0)),
            pl.BlockSpec((H, H), lambda i: (0, 0)),
            pl.BlockSpec((H, OUT), lambda i: (0, 0)),
            pl.BlockSpec((1, OUT), lambda i: (0, 0)),
        ],
        out_specs=pl.BlockSpec((blk, OUT), lambda i: (i, 0)),
        out_shape=jax.ShapeDtypeStruct((E, OUT), jnp.float32),
    )(e1, d, c2, w1e, w2, b2)


def kernel(x, edge_index, edge_attr, u, eb_W, eb_b, nb_W, nb_b, gb_W, gb_b,
           dec_W1, dec_b1, dec_W2, dec_b2):
    senders = edge_index[0]
    receivers = edge_index[1]

    wea = eb_W[0:16]
    wea2 = jnp.zeros((32, H2), jnp.float32)
    wea2 = wea2.at[0:16, 0:H].set(wea).at[16:32, H:H2].set(wea)
    ws = eb_W[16:16 + D_NODE]
    wr = eb_W[16 + D_NODE:16 + 2 * D_NODE]
    wu = eb_W[16 + 2 * D_NODE:]
    wa = nb_W[0:H]
    wx = nb_W[H:H + D_NODE]
    wun = nb_W[H + D_NODE:]
    gbe = gb_W[0:H]
    gbn = gb_W[H:2 * H]
    gbu = gb_W[2 * H:]
    w1e = dec_W1[0:H]
    w1sr = jnp.concatenate([dec_W1[H:2 * H], dec_W1[2 * H:3 * H]], axis=1)
    w1g = dec_W1[3 * H:]

    u2 = u.reshape(1, H)
    ebb = eb_b.reshape(1, H)
    nbb = nb_b.reshape(1, H)
    gbb = gb_b.reshape(1, H)
    db1 = dec_b1.reshape(1, H)
    b2 = dec_b2.reshape(1, OUT)
    zeros_n = jnp.zeros((N, H), jnp.float32)
    snd3 = senders.reshape(NW, NCHUNK, CH)
    rcv3 = receivers.reshape(NW, NCHUNK, CH)
    attr2 = edge_attr.reshape(E // 2, 32)

    xs, xr = _proj_nodes(x, ws, wr)
    ea1 = _edge_base(attr2, wea2, u2, wu, ebb).reshape(E, H)
    e1, aggp = _sc_edge_pass1(ea1, xs, xr, snd3, rcv3, zeros_n)
    nsr, c2 = _node_block(aggp[:N], aggp[N:], x, u2, wa, wx, wun, nbb,
                          gbe, gbn, gbu, gbb, w1sr, w1g, db1)
    d = _sc_edge_pass2(nsr, snd3, rcv3)
    return _decoder(e1, d, c2, w1e, dec_W2, b2)
